# baseline (speedup 1.0000x reference)
import numpy as np
import jax
import jax.numpy as jnp
from jax import lax
from jax.experimental import pallas as pl
from jax.experimental.pallas import tpu as pltpu
from jax.experimental.pallas import tpu_sc as plsc

_B = 4096
_F = 26
_V = 100000
_D = 32
_NF1 = _F + 1
_NIDX = _B * _F
_WIN = 128
_BBLK = 512

_LI, _LJ = np.tril_indices(_NF1, -1)


_NW = 32
_NR = _F * _D
_RPW = _NR // _NW


def _sc_gather_t(tab_t, lS_i):
    mesh = plsc.VectorSubcoreMesh(core_axis_name="core", subcore_axis_name="subcore")

    @pl.kernel(
        out_type=jax.ShapeDtypeStruct((_NR, _B), jnp.float32),
        mesh=mesh,
        compiler_params=pltpu.CompilerParams(needs_layout_passes=False),
        scratch_types=[
            pltpu.VMEM((1, _V), jnp.float32),
            pltpu.VMEM((1, _B), jnp.int32),
            pltpu.VMEM((2, 1, _B), jnp.float32),
            pltpu.SemaphoreType.DMA,
            pltpu.SemaphoreType.DMA,
        ],
    )
    def k(tab_hbm, i_hbm, o_hbm, row_v, idx_v, out_v, rsem, osem):
        w = jax.lax.axis_index("subcore") * 2 + jax.lax.axis_index("core")
        stores = [None, None]
        for j in range(_RPW):
            r = w * _RPW + j
            f = jax.lax.div(r, _D)
            rcp = pltpu.async_copy(tab_hbm.at[pl.ds(r, 1)], row_v, rsem)
            pltpu.sync_copy(i_hbm.at[pl.ds(f, 1)], idx_v)
            rcp.wait()
            ob = out_v.at[j % 2]
            zero = jnp.zeros((16,), jnp.int32)

            @pl.loop(0, _B // 16, unroll=8)
            def _(i):
                v = idx_v[0, pl.ds(i * 16, 16)]
                ob[0, pl.ds(i * 16, 16)] = plsc.load_gather(row_v, [zero, v])

            if stores[j % 2] is not None:
                stores[j % 2].wait()
            stores[j % 2] = pltpu.async_copy(ob, o_hbm.at[pl.ds(r, 1)], osem)
        for cp in stores:
            if cp is not None:
                cp.wait()

    return k(tab_t, lS_i)


def _tc_body(x_ref, ly_ref, w0t, b0, w1t, b1, w2t, b2, wx, wz, tb0, tw1t, tb1,
             tw2t, tb2, o_ref):
    x = x_ref[...]
    h = jnp.maximum(jnp.dot(x, w0t[...], preferred_element_type=jnp.float32) + b0[...], 0.0)
    h = jnp.maximum(jnp.dot(h, w1t[...], preferred_element_type=jnp.float32) + b1[...], 0.0)
    x3 = jnp.maximum(jnp.dot(h, w2t[...], preferred_element_type=jnp.float32) + b2[...], 0.0)
    ly = ly_ref[...].T
    t3 = jnp.concatenate([x3[:, None, :], ly.reshape(_BBLK, _F, _D)], axis=1)
    z = lax.dot_general(t3, t3, (((2,), (2,)), ((0,), (0,))),
                        preferred_element_type=jnp.float32)
    zf = z.reshape(_BBLK, _NF1 * _NF1)
    y = (jnp.dot(x3, wx[...], preferred_element_type=jnp.float32)
         + jnp.dot(zf, wz[...], preferred_element_type=jnp.float32) + tb0[...])
    y = jnp.maximum(y, 0.0)
    y = jnp.maximum(jnp.dot(y, tw1t[...], preferred_element_type=jnp.float32) + tb1[...], 0.0)
    y = jnp.dot(y, tw2t[...], preferred_element_type=jnp.float32) + tb2[...]
    o_ref[...] = 1.0 / (1.0 + jnp.exp(-y))


def _tc_dense(dense_x, ly, w0t, b0, w1t, b1, w2t, b2, wx, wz, tb0, tw1t, tb1,
              tw2t, tb2):
    nblk = _B // _BBLK
    full = lambda shape: pl.BlockSpec(shape, lambda i: (0, 0))
    return pl.pallas_call(
        _tc_body,
        grid=(nblk,),
        in_specs=[
            pl.BlockSpec((_BBLK, 13), lambda i: (i, 0)),
            pl.BlockSpec((_NR, _BBLK), lambda i: (0, i)),
            full((13, 512)), full((1, 512)),
            full((512, 256)), full((1, 256)),
            full((256, 32)), full((1, 32)),
            full((32, 512)), full((_NF1 * _NF1, 512)), full((1, 512)),
            full((512, 256)), full((1, 256)),
            full((256, 1)), full((1, 1)),
        ],
        out_specs=pl.BlockSpec((_BBLK, 1), lambda i: (i, 0)),
        out_shape=jax.ShapeDtypeStruct((_B, 1), jnp.float32),
    )(dense_x, ly, w0t, b0, w1t, b1, w2t, b2, wx, wz, tb0, tw1t, tb1, tw2t, tb2)


def kernel(dense_x, lS_i, emb_tables, bot_w0, bot_b0, bot_w1, bot_b1, bot_w2,
           bot_b2, top_w0, top_b0, top_w1, top_b1, top_w2, top_b2):
    tab_t = jnp.swapaxes(emb_tables, 1, 2).reshape(_NR, _V)
    ly = _sc_gather_t(tab_t, lS_i.astype(jnp.int32))

    w0t, w1t, w2t = bot_w0.T, bot_w1.T, bot_w2.T
    tw1t, tw2t = top_w1.T, top_w2.T
    wx = top_w0[:, :_D].T
    pair_pos = _LI * _NF1 + _LJ
    wz = jnp.zeros((_NF1 * _NF1, 512), jnp.float32).at[pair_pos, :].set(
        top_w0[:, _D:].T)

    return _tc_dense(
        dense_x, ly, w0t, bot_b0[None, :], w1t, bot_b1[None, :], w2t,
        bot_b2[None, :], wx, wz, top_b0[None, :], tw1t, top_b1[None, :], tw2t,
        top_b2[None, :])


# speedup vs baseline: 1.0754x; 1.0754x over previous
import numpy as np
import jax
import jax.numpy as jnp
from jax import lax
from jax.experimental import pallas as pl
from jax.experimental.pallas import tpu as pltpu
from jax.experimental.pallas import tpu_sc as plsc

_B = 4096
_F = 26
_V = 100000
_D = 32
_NF1 = _F + 1
_NIDX = _B * _F
_WIN = 128
_BBLK = 512

_LI, _LJ = np.tril_indices(_NF1, -1)


_NW = 32
_NR = _F * _D
_RPW = _NR // _NW


def _sc_gather_t(tab_t, lS_i):
    mesh = plsc.VectorSubcoreMesh(core_axis_name="core", subcore_axis_name="subcore")

    @pl.kernel(
        out_type=jax.ShapeDtypeStruct((_NR, _B), jnp.float32),
        mesh=mesh,
        compiler_params=pltpu.CompilerParams(needs_layout_passes=False),
        scratch_types=[
            pltpu.VMEM((1, _V), jnp.float32),
            pltpu.VMEM((1, _B), jnp.int32),
            pltpu.VMEM((2, 1, _B), jnp.float32),
            pltpu.SemaphoreType.DMA,
            pltpu.SemaphoreType.DMA,
        ],
    )
    def k(tab_hbm, i_hbm, o_hbm, row_v, idx_v, out_v, rsem, osem):
        w = jax.lax.axis_index("subcore") * 2 + jax.lax.axis_index("core")
        stores = [None, None]
        for j in range(_RPW):
            r = w * _RPW + j
            f = jax.lax.div(r, _D)
            rcp = pltpu.async_copy(tab_hbm.at[pl.ds(r, 1)], row_v, rsem)
            pltpu.sync_copy(i_hbm.at[pl.ds(f, 1)], idx_v)
            rcp.wait()
            ob = out_v.at[j % 2]
            zero = jnp.zeros((16,), jnp.int32)

            @pl.loop(0, _B // 16)
            def _(i):
                v = idx_v[0, pl.ds(i * 16, 16)]
                ob[0, pl.ds(i * 16, 16)] = plsc.load_gather(row_v, [zero, v])

            if stores[j % 2] is not None:
                stores[j % 2].wait()
            stores[j % 2] = pltpu.async_copy(ob, o_hbm.at[pl.ds(r, 1)], osem)
        for cp in stores:
            if cp is not None:
                cp.wait()

    return k(tab_t, lS_i)


def _tc_body(x_ref, ly_ref, w0t, b0, w1t, b1, w2t, b2, wx, wz, tb0, tw1t, tb1,
             tw2t, tb2, o_ref):
    x = x_ref[...]
    h = jnp.maximum(jnp.dot(x, w0t[...], preferred_element_type=jnp.float32) + b0[...], 0.0)
    h = jnp.maximum(jnp.dot(h, w1t[...], preferred_element_type=jnp.float32) + b1[...], 0.0)
    x3 = jnp.maximum(jnp.dot(h, w2t[...], preferred_element_type=jnp.float32) + b2[...], 0.0)
    ly = ly_ref[...].T
    t3 = jnp.concatenate([x3[:, None, :], ly.reshape(_BBLK, _F, _D)], axis=1)
    z = lax.dot_general(t3, t3, (((2,), (2,)), ((0,), (0,))),
                        preferred_element_type=jnp.float32)
    zf = z.reshape(_BBLK, _NF1 * _NF1)
    y = (jnp.dot(x3, wx[...], preferred_element_type=jnp.float32)
         + jnp.dot(zf, wz[...], preferred_element_type=jnp.float32) + tb0[...])
    y = jnp.maximum(y, 0.0)
    y = jnp.maximum(jnp.dot(y, tw1t[...], preferred_element_type=jnp.float32) + tb1[...], 0.0)
    y = jnp.dot(y, tw2t[...], preferred_element_type=jnp.float32) + tb2[...]
    o_ref[...] = 1.0 / (1.0 + jnp.exp(-y))


def _tc_dense(dense_x, ly, w0t, b0, w1t, b1, w2t, b2, wx, wz, tb0, tw1t, tb1,
              tw2t, tb2):
    nblk = _B // _BBLK
    full = lambda shape: pl.BlockSpec(shape, lambda i: (0, 0))
    return pl.pallas_call(
        _tc_body,
        grid=(nblk,),
        in_specs=[
            pl.BlockSpec((_BBLK, 13), lambda i: (i, 0)),
            pl.BlockSpec((_NR, _BBLK), lambda i: (0, i)),
            full((13, 512)), full((1, 512)),
            full((512, 256)), full((1, 256)),
            full((256, 32)), full((1, 32)),
            full((32, 512)), full((_NF1 * _NF1, 512)), full((1, 512)),
            full((512, 256)), full((1, 256)),
            full((256, 1)), full((1, 1)),
        ],
        out_specs=pl.BlockSpec((_BBLK, 1), lambda i: (i, 0)),
        out_shape=jax.ShapeDtypeStruct((_B, 1), jnp.float32),
    )(dense_x, ly, w0t, b0, w1t, b1, w2t, b2, wx, wz, tb0, tw1t, tb1, tw2t, tb2)


def kernel(dense_x, lS_i, emb_tables, bot_w0, bot_b0, bot_w1, bot_b1, bot_w2,
           bot_b2, top_w0, top_b0, top_w1, top_b1, top_w2, top_b2):
    tab_t = jnp.swapaxes(emb_tables, 1, 2).reshape(_NR, _V)
    ly = _sc_gather_t(tab_t, lS_i.astype(jnp.int32))

    w0t, w1t, w2t = bot_w0.T, bot_w1.T, bot_w2.T
    tw1t, tw2t = top_w1.T, top_w2.T
    wx = top_w0[:, :_D].T
    pair_pos = _LI * _NF1 + _LJ
    wz = jnp.zeros((_NF1 * _NF1, 512), jnp.float32).at[pair_pos, :].set(
        top_w0[:, _D:].T)

    return _tc_dense(
        dense_x, ly, w0t, bot_b0[None, :], w1t, bot_b1[None, :], w2t,
        bot_b2[None, :], wx, wz, top_b0[None, :], tw1t, top_b1[None, :], tw2t,
        top_b2[None, :])
